# baseline (speedup 1.0000x reference)
import jax
import jax.numpy as jnp
from jax import lax
from jax.experimental import pallas as pl
from jax.experimental.pallas import tpu as pltpu
from jax.experimental.pallas import tpu_sc as plsc

VOCAB = 1000000
DIM = 50
ROWP = 128
B = 4096
L = 200

NC = 2
NS = 16
NW = NC * NS

N_TOK = B * L
PER_W = N_TOK // NW
CHUNK = 128
NCH = PER_W // CHUNK

VB = 512


def _tc_transpose(tab_t):
    def body(in_ref, out_ref):
        xt = jnp.swapaxes(in_ref[...], 0, 1)
        out_ref[...] = jnp.concatenate(
            [xt, jnp.zeros((VB, ROWP - DIM), jnp.float32)], axis=1)

    grid = (VOCAB + VB - 1) // VB
    return pl.pallas_call(
        body,
        grid=(grid,),
        in_specs=[pl.BlockSpec((DIM, VB), lambda i: (0, i))],
        out_specs=pl.BlockSpec((VB, ROWP), lambda i: (i, 0)),
        out_shape=jax.ShapeDtypeStruct((VOCAB, ROWP), jnp.float32),
    )(tab_t)


def _sc_body(idx_hbm, table_hbm, out_hbm, idx_v, rows_v, gsem):
    wid = lax.axis_index("s") * NC + lax.axis_index("c")
    base = wid * PER_W
    pltpu.sync_copy(idx_hbm.at[wid], idx_v)

    def step(j, carry):
        pltpu.async_copy(table_hbm.at[idx_v.at[j]], rows_v, gsem).wait()
        pltpu.sync_copy(rows_v, out_hbm.at[pl.ds(base + j * CHUNK, CHUNK)])
        return carry

    lax.fori_loop(0, NCH, step, 0)


@jax.jit
def kernel(indices, table):
    idx = indices.reshape(NW, NCH, CHUNK).astype(jnp.int32)
    tab128 = _tc_transpose(jnp.swapaxes(table, 0, 1))
    mesh = plsc.VectorSubcoreMesh(core_axis_name="c", subcore_axis_name="s")
    out = pl.kernel(
        _sc_body,
        mesh=mesh,
        compiler_params=pltpu.CompilerParams(
            use_tc_tiling_on_sc=True, skip_device_barrier=True),
        out_type=jax.ShapeDtypeStruct((N_TOK, ROWP), jnp.float32),
        scratch_types=[
            pltpu.VMEM((NCH, CHUNK), jnp.int32),
            pltpu.VMEM((CHUNK, ROWP), jnp.float32),
            pltpu.SemaphoreType.DMA,
        ],
    )(idx, tab128)
    return out[:, :DIM].reshape(B, L, DIM)


# speedup vs baseline: 1.5337x; 1.5337x over previous
import jax
import jax.numpy as jnp
from jax import lax
from jax.experimental import pallas as pl
from jax.experimental.pallas import tpu as pltpu
from jax.experimental.pallas import tpu_sc as plsc

VOCAB = 1000000
DIM = 50
ROWP = 128
DIMP = 56
B = 4096
L = 200

NC = 2
NS = 16
NW = NC * NS

N_TOK = B * L
PER_W = N_TOK // NW
CHUNK = 128
NCH = PER_W // CHUNK

VB = 1024


def _tc_transpose(tab_t):
    def body(in_ref, out_ref):
        xt = jnp.swapaxes(in_ref[...], 0, 1)
        out_ref[...] = jnp.concatenate(
            [xt, jnp.zeros((VB, ROWP - DIM), jnp.float32)], axis=1)

    grid = (VOCAB + VB - 1) // VB
    return pl.pallas_call(
        body,
        grid=(grid,),
        in_specs=[pl.BlockSpec((DIM, VB), lambda i: (0, i))],
        out_specs=pl.BlockSpec((VB, ROWP), lambda i: (i, 0)),
        out_shape=jax.ShapeDtypeStruct((VOCAB, ROWP), jnp.float32),
    )(tab_t)


NBUF = 4


def _sc_body(idx_hbm, table_hbm, out_hbm, idx_v, rows_v,
             g0, g1, g2, g3, w0, w1, w2, w3):
    gsem = (g0, g1, g2, g3)
    wsem = (w0, w1, w2, w3)
    wid = lax.axis_index("s") * NC + lax.axis_index("c")
    base = wid * PER_W
    pltpu.sync_copy(idx_hbm.at[wid], idx_v)

    for k in range(NBUF):
        pltpu.async_copy(table_hbm.at[idx_v.at[k]], rows_v.at[k], gsem[k])

    def round_(q, carry):
        for k in range(NBUF):
            j = q * NBUF + k
            pltpu.make_async_copy(table_hbm.at[idx_v.at[j]],
                                  rows_v.at[k], gsem[k]).wait()
            pltpu.async_copy(rows_v.at[k],
                             out_hbm.at[pl.ds(base + j * CHUNK, CHUNK)],
                             wsem[k])

            @pl.when(j + NBUF < NCH)
            def _():
                pltpu.make_async_copy(
                    rows_v.at[k],
                    out_hbm.at[pl.ds(base + j * CHUNK, CHUNK)],
                    wsem[k]).wait()
                pltpu.async_copy(table_hbm.at[idx_v.at[j + NBUF]],
                                 rows_v.at[k], gsem[k])
        return carry

    lax.fori_loop(0, NCH // NBUF, round_, 0)

    for k in range(NBUF):
        j = NCH - NBUF + k
        pltpu.make_async_copy(rows_v.at[k],
                              out_hbm.at[pl.ds(base + j * CHUNK, CHUNK)],
                              wsem[k]).wait()


@jax.jit
def kernel(indices, table):
    idx = indices.reshape(NW, NCH, CHUNK).astype(jnp.int32)
    tab128 = _tc_transpose(jnp.swapaxes(table, 0, 1))
    mesh = plsc.VectorSubcoreMesh(core_axis_name="c", subcore_axis_name="s")
    out = pl.kernel(
        _sc_body,
        mesh=mesh,
        compiler_params=pltpu.CompilerParams(use_tc_tiling_on_sc=True),
        out_type=jax.ShapeDtypeStruct((N_TOK, ROWP), jnp.float32),
        scratch_types=[
            pltpu.VMEM((NCH, CHUNK), jnp.int32),
            pltpu.VMEM((NBUF, CHUNK, ROWP), jnp.float32),
        ] + [pltpu.SemaphoreType.DMA] * (2 * NBUF),
    )(idx, tab128)
    return out[:, :DIM].reshape(B, L, DIM)


# speedup vs baseline: 1.9426x; 1.2666x over previous
import jax
import jax.numpy as jnp
from jax import lax
from jax.experimental import pallas as pl
from jax.experimental.pallas import tpu as pltpu
from jax.experimental.pallas import tpu_sc as plsc

VOCAB = 1000000
DIM = 50
ROWP = 128
DIMP = 56
B = 4096
L = 200

NC = 2
NS = 16
NW = NC * NS

N_TOK = B * L
PER_W = N_TOK // NW
CHUNK = 128
NCH = PER_W // CHUNK

VB = 2048


def _tc_transpose(tab_t):
    def body(in_ref, out_ref):
        xt = jnp.swapaxes(in_ref[...], 0, 1)
        out_ref[...] = jnp.concatenate(
            [xt, jnp.zeros((VB, ROWP - DIM), jnp.float32)], axis=1)

    grid = (VOCAB + VB - 1) // VB
    return pl.pallas_call(
        body,
        grid=(grid,),
        in_specs=[pl.BlockSpec((DIM, VB), lambda i: (0, i))],
        out_specs=pl.BlockSpec((VB, ROWP), lambda i: (i, 0)),
        out_shape=jax.ShapeDtypeStruct((VOCAB, ROWP), jnp.float32),
    )(tab_t)


NBUF = 5


def _sc_body(idx_hbm, table_hbm, out_hbm, idx_v, rows_v, *sems):
    gsem = sems[:NBUF]
    wsem = sems[NBUF:]
    wid = lax.axis_index("s") * NC + lax.axis_index("c")
    base = wid * PER_W
    pltpu.sync_copy(idx_hbm.at[wid], idx_v)

    for k in range(NBUF):
        pltpu.async_copy(table_hbm.at[idx_v.at[k]], rows_v.at[k], gsem[k])

    def round_(q, carry):
        for k in range(NBUF):
            j = q * NBUF + k
            pltpu.make_async_copy(table_hbm.at[idx_v.at[j]],
                                  rows_v.at[k], gsem[k]).wait()
            pltpu.async_copy(rows_v.at[k],
                             out_hbm.at[pl.ds(base + j * CHUNK, CHUNK)],
                             wsem[k])

            @pl.when(j + NBUF < NCH)
            def _():
                pltpu.make_async_copy(
                    rows_v.at[k],
                    out_hbm.at[pl.ds(base + j * CHUNK, CHUNK)],
                    wsem[k]).wait()
                pltpu.async_copy(table_hbm.at[idx_v.at[j + NBUF]],
                                 rows_v.at[k], gsem[k])
        return carry

    lax.fori_loop(0, NCH // NBUF, round_, 0)

    for k in range(NBUF):
        j = NCH - NBUF + k
        pltpu.make_async_copy(rows_v.at[k],
                              out_hbm.at[pl.ds(base + j * CHUNK, CHUNK)],
                              wsem[k]).wait()


@jax.jit
def kernel(indices, table):
    idx = indices.reshape(NW, NCH, CHUNK).astype(jnp.int32)
    tab128 = _tc_transpose(jnp.swapaxes(table, 0, 1))
    mesh = plsc.VectorSubcoreMesh(core_axis_name="c", subcore_axis_name="s")
    out = pl.kernel(
        _sc_body,
        mesh=mesh,
        compiler_params=pltpu.CompilerParams(use_tc_tiling_on_sc=True),
        out_type=jax.ShapeDtypeStruct((N_TOK, ROWP), jnp.float32),
        scratch_types=[
            pltpu.VMEM((NCH, CHUNK), jnp.int32),
            pltpu.VMEM((NBUF, CHUNK, ROWP), jnp.float32),
        ] + [pltpu.SemaphoreType.DMA] * (2 * NBUF),
    )(idx, tab128)
    return out[:, :DIM].reshape(B, L, DIM)


# speedup vs baseline: 2.2413x; 1.1537x over previous
import jax
import jax.numpy as jnp
from jax import lax
from jax.experimental import pallas as pl
from jax.experimental.pallas import tpu as pltpu
from jax.experimental.pallas import tpu_sc as plsc

VOCAB = 1000000
DIM = 50
ROWP = 128
DIMP = 56
B = 4096
L = 200

NC = 2
NS = 16
NW = NC * NS

N_TOK = B * L
PER_W = N_TOK // NW
CHUNK = 128
NCH = PER_W // CHUNK

VB = 4096


def _tc_transpose(tab_t):
    def body(in_ref, out_ref):
        xt = jnp.swapaxes(in_ref[...], 0, 1)
        out_ref[...] = jnp.concatenate(
            [xt, jnp.zeros((VB, ROWP - DIM), jnp.float32)], axis=1)

    grid = (VOCAB + VB - 1) // VB
    return pl.pallas_call(
        body,
        grid=(grid,),
        in_specs=[pl.BlockSpec((DIM, VB), lambda i: (0, i))],
        out_specs=pl.BlockSpec((VB, ROWP), lambda i: (i, 0)),
        out_shape=jax.ShapeDtypeStruct((VOCAB, ROWP), jnp.float32),
    )(tab_t)


NBUF = 5


def _sc_body(idx_hbm, table_hbm, out_hbm, idx_v, rows_v, *sems):
    gsem = sems[:NBUF]
    wsem = sems[NBUF:]
    wid = lax.axis_index("s") * NC + lax.axis_index("c")
    base = wid * PER_W
    pltpu.sync_copy(idx_hbm.at[wid], idx_v)

    for k in range(NBUF):
        pltpu.async_copy(table_hbm.at[idx_v.at[k]], rows_v.at[k], gsem[k])

    def round_(q, carry):
        for k in range(NBUF):
            j = q * NBUF + k
            pltpu.make_async_copy(table_hbm.at[idx_v.at[j]],
                                  rows_v.at[k], gsem[k]).wait()
            pltpu.async_copy(rows_v.at[k],
                             out_hbm.at[pl.ds(base + j * CHUNK, CHUNK)],
                             wsem[k])

            @pl.when(j + NBUF < NCH)
            def _():
                pltpu.make_async_copy(
                    rows_v.at[k],
                    out_hbm.at[pl.ds(base + j * CHUNK, CHUNK)],
                    wsem[k]).wait()
                pltpu.async_copy(table_hbm.at[idx_v.at[j + NBUF]],
                                 rows_v.at[k], gsem[k])
        return carry

    lax.fori_loop(0, NCH // NBUF, round_, 0)

    for k in range(NBUF):
        j = NCH - NBUF + k
        pltpu.make_async_copy(rows_v.at[k],
                              out_hbm.at[pl.ds(base + j * CHUNK, CHUNK)],
                              wsem[k]).wait()


@jax.jit
def kernel(indices, table):
    idx = indices.reshape(NW, NCH, CHUNK).astype(jnp.int32)
    tab128 = _tc_transpose(jnp.swapaxes(table, 0, 1))
    mesh = plsc.VectorSubcoreMesh(core_axis_name="c", subcore_axis_name="s")
    out = pl.kernel(
        _sc_body,
        mesh=mesh,
        compiler_params=pltpu.CompilerParams(use_tc_tiling_on_sc=True),
        out_type=jax.ShapeDtypeStruct((N_TOK, ROWP), jnp.float32),
        scratch_types=[
            pltpu.VMEM((NCH, CHUNK), jnp.int32),
            pltpu.VMEM((NBUF, CHUNK, ROWP), jnp.float32),
        ] + [pltpu.SemaphoreType.DMA] * (2 * NBUF),
    )(idx, tab128)
    return out[:, :DIM].reshape(B, L, DIM)


# speedup vs baseline: 2.2425x; 1.0005x over previous
import jax
import jax.numpy as jnp
from jax import lax
from jax.experimental import pallas as pl
from jax.experimental.pallas import tpu as pltpu
from jax.experimental.pallas import tpu_sc as plsc

VOCAB = 1000000
DIM = 50
ROWP = 128
B = 4096
L = 200

NC = 2
NS = 16
NW = NC * NS

N_TOK = B * L
PER_W = N_TOK // NW
CHUNK = 128
NCH = PER_W // CHUNK

VB = 4096


def _tc_transpose(tab_t):
    def body(in_ref, out_ref):
        xt = jnp.swapaxes(in_ref[...], 0, 1)
        out_ref[...] = jnp.concatenate(
            [xt, jnp.zeros((VB, ROWP - DIM), jnp.float32)], axis=1)

    grid = (VOCAB + VB - 1) // VB
    return pl.pallas_call(
        body,
        grid=(grid,),
        in_specs=[pl.BlockSpec((DIM, VB), lambda i: (0, i))],
        out_specs=pl.BlockSpec((VB, ROWP), lambda i: (i, 0)),
        out_shape=jax.ShapeDtypeStruct((VOCAB, ROWP), jnp.float32),
    )(tab_t)


NBUF = 5


def _sc_body(idx_hbm, table_hbm, out_hbm, idx_v, rows_v, *sems):
    gsem = sems[:NBUF]
    wsem = sems[NBUF:]
    wid = lax.axis_index("s") * NC + lax.axis_index("c")
    base = wid * PER_W
    pltpu.sync_copy(idx_hbm.at[wid], idx_v)

    for k in range(NBUF):
        pltpu.async_copy(table_hbm.at[idx_v.at[k]], rows_v.at[k], gsem[k])

    def round_(q, carry):
        for k in range(NBUF):
            j = q * NBUF + k
            pltpu.make_async_copy(table_hbm.at[idx_v.at[j]],
                                  rows_v.at[k], gsem[k]).wait()
            pltpu.async_copy(rows_v.at[k],
                             out_hbm.at[pl.ds(base + j * CHUNK, CHUNK)],
                             wsem[k])

            @pl.when(j + NBUF < NCH)
            def _():
                pltpu.make_async_copy(
                    rows_v.at[k],
                    out_hbm.at[pl.ds(base + j * CHUNK, CHUNK)],
                    wsem[k]).wait()
                pltpu.async_copy(table_hbm.at[idx_v.at[j + NBUF]],
                                 rows_v.at[k], gsem[k])
        return carry

    lax.fori_loop(0, NCH // NBUF, round_, 0)

    for k in range(NBUF):
        j = NCH - NBUF + k
        pltpu.make_async_copy(rows_v.at[k],
                              out_hbm.at[pl.ds(base + j * CHUNK, CHUNK)],
                              wsem[k]).wait()


@jax.jit
def kernel(indices, table):
    idx = indices.reshape(NW, NCH, CHUNK).astype(jnp.int32)
    tab128 = _tc_transpose(jnp.swapaxes(table, 0, 1))
    mesh = plsc.VectorSubcoreMesh(core_axis_name="c", subcore_axis_name="s")
    out = pl.kernel(
        _sc_body,
        mesh=mesh,
        compiler_params=pltpu.CompilerParams(use_tc_tiling_on_sc=True),
        out_type=jax.ShapeDtypeStruct((N_TOK, ROWP), jnp.float32),
        scratch_types=[
            pltpu.VMEM((NCH, CHUNK), jnp.int32),
            pltpu.VMEM((NBUF, CHUNK, ROWP), jnp.float32),
        ] + [pltpu.SemaphoreType.DMA] * (2 * NBUF),
    )(idx, tab128)
    return out[:, :DIM].reshape(B, L, DIM)
